# initial kernel scaffold (unmeasured)
import jax
import jax.numpy as jnp
from jax import lax
from jax.experimental import pallas as pl
from jax.experimental.pallas import tpu as pltpu


def kernel(
    x,
):
    def body(*refs):
        pass

    out_shape = jax.ShapeDtypeStruct(..., jnp.float32)
    return pl.pallas_call(body, out_shape=out_shape)(...)


# baseline (device time: 10238 ns/iter reference)
import jax
import jax.numpy as jnp
from jax import lax
from jax.experimental import pallas as pl
from jax.experimental.pallas import tpu as pltpu

N_DEV = 32


def kernel(x):
    m, n = x.shape

    def body(x_ref, out_ref, halo_ref, send_sems, recv_sems):
        my = lax.axis_index("i")
        left = lax.rem(my - 1 + N_DEV, N_DEV)
        right = lax.rem(my + 1, N_DEV)

        barrier_sem = pltpu.get_barrier_semaphore()
        for nbr in (left, right):
            pl.semaphore_signal(
                barrier_sem, inc=1,
                device_id=(nbr,), device_id_type=pl.DeviceIdType.MESH,
            )
        pl.semaphore_wait(barrier_sem, 2)

        send_left = pltpu.make_async_remote_copy(
            src_ref=x_ref.at[pl.ds(0, 1), :],
            dst_ref=halo_ref.at[pl.ds(1, 1), :],
            send_sem=send_sems.at[0],
            recv_sem=recv_sems.at[1],
            device_id=(left,),
            device_id_type=pl.DeviceIdType.MESH,
        )
        send_right = pltpu.make_async_remote_copy(
            src_ref=x_ref.at[pl.ds(m - 1, 1), :],
            dst_ref=halo_ref.at[pl.ds(0, 1), :],
            send_sem=send_sems.at[1],
            recv_sem=recv_sems.at[0],
            device_id=(right,),
            device_id_type=pl.DeviceIdType.MESH,
        )
        send_left.start()
        send_right.start()

        xv = x_ref[:, :]
        out_ref[pl.ds(1, m - 2), :] = (
            0.25 * xv[0 : m - 2, :] + 0.5 * xv[1 : m - 1, :] + 0.25 * xv[2:m, :]
        )

        send_right.wait_recv()

        @pl.when(my == 0)
        def _():
            out_ref[pl.ds(0, 1), :] = xv[0:1, :]

        @pl.when(my != 0)
        def _():
            out_ref[pl.ds(0, 1), :] = (
                0.25 * halo_ref[pl.ds(0, 1), :]
                + 0.5 * xv[0:1, :]
                + 0.25 * xv[1:2, :]
            )

        send_left.wait_recv()

        @pl.when(my == N_DEV - 1)
        def _():
            out_ref[pl.ds(m - 1, 1), :] = xv[m - 1 : m, :]

        @pl.when(my != N_DEV - 1)
        def _():
            out_ref[pl.ds(m - 1, 1), :] = (
                0.25 * xv[m - 2 : m - 1, :]
                + 0.5 * xv[m - 1 : m, :]
                + 0.25 * halo_ref[pl.ds(1, 1), :]
            )

        send_left.wait_send()
        send_right.wait_send()

    return pl.pallas_call(
        body,
        out_shape=jax.ShapeDtypeStruct((m, n), x.dtype),
        in_specs=[pl.BlockSpec(memory_space=pltpu.VMEM)],
        out_specs=pl.BlockSpec(memory_space=pltpu.VMEM),
        scratch_shapes=[
            pltpu.VMEM((2, n), x.dtype),
            pltpu.SemaphoreType.DMA((2,)),
            pltpu.SemaphoreType.DMA((2,)),
        ],
        compiler_params=pltpu.CompilerParams(collective_id=0),
    )(x)


# device time: 6134 ns/iter; 1.6691x vs baseline; 1.6691x over previous
import jax
import jax.numpy as jnp
from jax import lax
from jax.experimental import pallas as pl
from jax.experimental.pallas import tpu as pltpu

N_DEV = 32


def kernel(x):
    m, n = x.shape

    def body(x_ref, out_ref, halo_ref, send_sems, recv_sems):
        my = lax.axis_index("i")
        has_left = my > 0
        has_right = my < N_DEV - 1
        left = lax.max(my - 1, 0)
        right = lax.min(my + 1, N_DEV - 1)

        barrier_sem = pltpu.get_barrier_semaphore()

        @pl.when(has_left)
        def _():
            pl.semaphore_signal(
                barrier_sem, inc=1,
                device_id=(left,), device_id_type=pl.DeviceIdType.MESH,
            )

        @pl.when(has_right)
        def _():
            pl.semaphore_signal(
                barrier_sem, inc=1,
                device_id=(right,), device_id_type=pl.DeviceIdType.MESH,
            )

        @pl.when(has_left & has_right)
        def _():
            pl.semaphore_wait(barrier_sem, 2)

        @pl.when(~(has_left & has_right))
        def _():
            pl.semaphore_wait(barrier_sem, 1)

        send_left = pltpu.make_async_remote_copy(
            src_ref=x_ref.at[pl.ds(0, 1), :],
            dst_ref=halo_ref.at[pl.ds(1, 1), :],
            send_sem=send_sems.at[0],
            recv_sem=recv_sems.at[1],
            device_id=(left,),
            device_id_type=pl.DeviceIdType.MESH,
        )
        send_right = pltpu.make_async_remote_copy(
            src_ref=x_ref.at[pl.ds(m - 1, 1), :],
            dst_ref=halo_ref.at[pl.ds(0, 1), :],
            send_sem=send_sems.at[1],
            recv_sem=recv_sems.at[0],
            device_id=(right,),
            device_id_type=pl.DeviceIdType.MESH,
        )

        @pl.when(has_left)
        def _():
            send_left.start()

        @pl.when(has_right)
        def _():
            send_right.start()

        xv = x_ref[:, :]
        out_ref[pl.ds(1, m - 2), :] = (
            0.25 * xv[0 : m - 2, :] + 0.5 * xv[1 : m - 1, :] + 0.25 * xv[2:m, :]
        )

        @pl.when(has_left)
        def _():
            send_right.wait_recv()
            out_ref[pl.ds(0, 1), :] = (
                0.25 * halo_ref[pl.ds(0, 1), :]
                + 0.5 * xv[0:1, :]
                + 0.25 * xv[1:2, :]
            )

        @pl.when(~has_left)
        def _():
            out_ref[pl.ds(0, 1), :] = xv[0:1, :]

        @pl.when(has_right)
        def _():
            send_left.wait_recv()
            out_ref[pl.ds(m - 1, 1), :] = (
                0.25 * xv[m - 2 : m - 1, :]
                + 0.5 * xv[m - 1 : m, :]
                + 0.25 * halo_ref[pl.ds(1, 1), :]
            )

        @pl.when(~has_right)
        def _():
            out_ref[pl.ds(m - 1, 1), :] = xv[m - 1 : m, :]

        @pl.when(has_left)
        def _():
            send_left.wait_send()

        @pl.when(has_right)
        def _():
            send_right.wait_send()

    return pl.pallas_call(
        body,
        out_shape=jax.ShapeDtypeStruct((m, n), x.dtype),
        in_specs=[pl.BlockSpec(memory_space=pltpu.VMEM)],
        out_specs=pl.BlockSpec(memory_space=pltpu.VMEM),
        scratch_shapes=[
            pltpu.VMEM((2, n), x.dtype),
            pltpu.SemaphoreType.DMA((2,)),
            pltpu.SemaphoreType.DMA((2,)),
        ],
        compiler_params=pltpu.CompilerParams(collective_id=0),
    )(x)
